# baseline (speedup 1.0000x reference)
import functools

import jax
import jax.numpy as jnp
from jax import lax
from jax.experimental import pallas as pl
from jax.experimental.pallas import tpu as pltpu

TEMPERATURE = 8.0
NUM_SAMPLED = 1024
EPS = 1e-06
_CHUNKS = 4


@functools.cache
def _gumbel(B, H, S):
    g = jax.random.gumbel(jax.random.key(42), (B, H, S), dtype=jnp.float32)
    return g.reshape(B * H, 1, S)


def _score_body(k_ref, q_ref, g_ref, tm_ref, o_ref, attn_ref):
    S = attn_ref.shape[1]
    CH = k_ref.shape[3]
    c = pl.program_id(1)
    kb = k_ref[0, 0].astype(jnp.bfloat16)
    qv = q_ref[0, 0, :, 0:1].astype(jnp.bfloat16)
    attn_ref[:, pl.ds(c * CH, CH)] = lax.dot_general(
        qv, kb, (((0,), (0,)), ((), ())),
        preferred_element_type=jnp.float32,
    ) / TEMPERATURE

    @pl.when(c == _CHUNKS - 1)
    def _tail():
        attn = attn_ref[...]
        m = jnp.max(attn)
        e = jnp.exp(attn - m) * tm_ref[0]
        se = jnp.sum(e)
        p = (e + EPS / S) / (se + EPS)
        o_ref[0] = jnp.log(p) + g_ref[0]


def _select_body(sc_ref, o_ref):
    R = sc_ref.shape[0]
    S = sc_ref.shape[2]
    sc = sc_ref[:, 0, :]

    ki = lax.bitcast_convert_type(sc, jnp.int32)
    t = ki ^ ((ki >> 31) & jnp.int32(0x7FFFFFFF))
    u = lax.bitcast_convert_type(t, jnp.uint32) ^ jnp.uint32(0x80000000)

    T = jnp.zeros((R, 1), dtype=jnp.uint32)
    for b in range(31, -1, -1):
        cand = T | jnp.uint32(1 << b)
        cnt = jnp.sum((u >= cand).astype(jnp.int32), axis=1, keepdims=True)
        T = jnp.where(cnt >= NUM_SAMPLED, cand, T)

    mask = (u >= T).astype(jnp.int32)
    rolled = pltpu.roll(mask, 1, 1)
    lane = lax.broadcasted_iota(jnp.int32, (R, S), 1)
    o_ref[:, 0, :] = jnp.where(lane == 0, 1, rolled)


def kernel(q, k, v, token_mask):
    B, H, S, D = q.shape
    R = B * H
    CH = S // _CHUNKS

    kT = jnp.swapaxes(k, 2, 3)
    qT = jnp.swapaxes(q, 2, 3)
    g = _gumbel(B, H, S)
    tm = token_mask.reshape(B, 1, S)

    scores = pl.pallas_call(
        _score_body,
        grid=(R, _CHUNKS),
        in_specs=[
            pl.BlockSpec((1, 1, D, CH), lambda r, c: (r // H, r % H, 0, c)),
            pl.BlockSpec((1, 1, D, 128), lambda r, c: (r // H, r % H, 0, 0)),
            pl.BlockSpec((1, 1, S), lambda r, c: (r, 0, 0)),
            pl.BlockSpec((1, 1, S), lambda r, c: (r // H, 0, 0)),
        ],
        out_specs=pl.BlockSpec((1, 1, S), lambda r, c: (r, 0, 0)),
        out_shape=jax.ShapeDtypeStruct((R, 1, S), jnp.float32),
        scratch_shapes=[pltpu.VMEM((1, S), jnp.float32)],
    )(kT, qT, g, tm)

    out = pl.pallas_call(
        _select_body,
        in_specs=[pl.BlockSpec((R, 1, S), lambda: (0, 0, 0))],
        out_specs=pl.BlockSpec((R, 1, S), lambda: (0, 0, 0)),
        out_shape=jax.ShapeDtypeStruct((R, 1, S), jnp.int32),
    )(scores)
    return out.reshape(B, H, S).astype(jnp.bool_)


# speedup vs baseline: 4.3348x; 4.3348x over previous
import functools

import jax
import jax.numpy as jnp
from jax import lax
from jax.experimental import pallas as pl
from jax.experimental.pallas import tpu as pltpu

TEMPERATURE = 8.0
NUM_SAMPLED = 1024
EPS = 1e-06
_G = 4


@functools.cache
def _gumbel(B, H, S):
    g = jax.random.gumbel(jax.random.key(42), (B, H, S), dtype=jnp.float32)
    return g.reshape(B * H // _G, _G, S)


def _score_body(k_ref, q_ref, g_ref, tm_ref, o_ref):
    S = k_ref.shape[2]
    kb = k_ref[0].astype(jnp.bfloat16)
    qa = q_ref[0].astype(jnp.bfloat16)
    attn = lax.dot_general(
        qa, kb, (((1,), (0,)), ((), ())),
        preferred_element_type=jnp.float32,
    ) / TEMPERATURE
    m = jnp.max(attn, axis=1, keepdims=True)
    e = jnp.exp(attn - m) * tm_ref[0]
    se = jnp.sum(e, axis=1, keepdims=True)
    p = (e + EPS / S) / (se + EPS)
    sc = jnp.log(p) + g_ref[0]

    ki = lax.bitcast_convert_type(sc, jnp.int32)
    t = ki ^ ((ki >> 31) & jnp.int32(0x7FFFFFFF))
    o_ref[0] = lax.bitcast_convert_type(t, jnp.uint32) ^ jnp.uint32(0x80000000)


def _select_body(u_ref, o_ref):
    NG, G, S = u_ref.shape
    R = NG * G
    u = u_ref[...].reshape(R, S)

    T = jnp.zeros((R, 1), dtype=jnp.uint32)
    for b in range(31, -1, -1):
        cand = T | jnp.uint32(1 << b)
        cnt = jnp.sum((u >= cand).astype(jnp.int32), axis=1, keepdims=True)
        T = jnp.where(cnt >= NUM_SAMPLED, cand, T)

    mask = (u >= T).astype(jnp.int32)
    rolled = pltpu.roll(mask, 1, 1)
    lane = lax.broadcasted_iota(jnp.int32, (R, S), 1)
    o_ref[...] = jnp.where(lane == 0, 1, rolled).reshape(NG, G, S)


def kernel(q, k, v, token_mask):
    B, H, S, D = q.shape
    R = B * H
    NG = R // _G
    GD = _G * D

    kG = jnp.swapaxes(k, 2, 3).reshape(NG, GD, S)
    q0 = q[:, :, 0, :].reshape(NG, _G, D)
    eye = jnp.eye(_G, dtype=q.dtype)
    qa = (q0[:, :, None, :] * eye[None, :, :, None]).reshape(NG, _G, GD)
    g = _gumbel(B, H, S)
    tm = token_mask.reshape(B, 1, S)

    keys = pl.pallas_call(
        _score_body,
        grid=(NG,),
        in_specs=[
            pl.BlockSpec((1, GD, S), lambda i: (i, 0, 0)),
            pl.BlockSpec((1, _G, GD), lambda i: (i, 0, 0)),
            pl.BlockSpec((1, _G, S), lambda i: (i, 0, 0)),
            pl.BlockSpec((1, 1, S), lambda i: (i * _G // H, 0, 0)),
        ],
        out_specs=pl.BlockSpec((1, _G, S), lambda i: (i, 0, 0)),
        out_shape=jax.ShapeDtypeStruct((NG, _G, S), jnp.uint32),
    )(kG, qa, g, tm)

    out = pl.pallas_call(
        _select_body,
        in_specs=[pl.BlockSpec((NG, _G, S), lambda: (0, 0, 0))],
        out_specs=pl.BlockSpec((NG, _G, S), lambda: (0, 0, 0)),
        out_shape=jax.ShapeDtypeStruct((NG, _G, S), jnp.int32),
    )(keys)
    return out.reshape(B, H, S).astype(jnp.bool_)
